# baseline (speedup 1.0000x reference)
import jax
import jax.numpy as jnp
from jax import lax
from jax.experimental import pallas as pl
from jax.experimental.pallas import tpu as pltpu
from jax.experimental.pallas import tpu_sc as plsc

_L = 16
_NC = 2
_NS = 16
_NW = _NC * _NS
_ROWS = 128
_N = 32768
_NV = _N // _L
_NG = _NV // _L
_RPW = _ROWS // _NW
_T = 12
_PAD = 256
_NEG = -3.0e38


def _tree_max(vs):
    while len(vs) > 1:
        vs = [jnp.maximum(vs[i], vs[i + 1]) for i in range(0, len(vs) - 1, 2)] + (
            [vs[-1]] if len(vs) % 2 else []
        )
    return vs[0]


def _sc_tau_body(x_hbm, tau_hbm, xva, xvb, av, gmax, clist, taubuf, sia, sib):
    cid = lax.axis_index("c")
    sid = lax.axis_index("s")
    wid = sid * _NC + cid
    row0 = wid * _RPW
    lane = lax.iota(jnp.int32, _L)
    zero = jnp.zeros((_L,), jnp.float32)
    negv = jnp.full((_L,), _NEG, jnp.float32)

    in_descs = [None] * _RPW
    in_descs[0] = pltpu.async_copy(x_hbm.at[row0], xva.at[pl.ds(0, _N)], sia)
    for j in range(_PAD // _L):
        xva[pl.ds(_N + j * _L, _L)] = negv
        xvb[pl.ds(_N + j * _L, _L)] = negv

    for r in range(_RPW):
        xv = xva if r % 2 == 0 else xvb
        in_descs[r].wait()
        if r + 1 < _RPW:
            nxv = xvb if r % 2 == 0 else xva
            nsi = sib if r % 2 == 0 else sia
            in_descs[r + 1] = pltpu.async_copy(
                x_hbm.at[row0 + r + 1], nxv.at[pl.ds(0, _N)], nsi
            )

        def p1(g, macc, xv=xv):
            vs = [xv[pl.ds(g * 256 + j * _L, _L)] for j in range(_L)]
            gm = _tree_max(vs)
            gmax[pl.ds(g * _L, _L)] = gm
            return jnp.maximum(macc, gm)

        macc = lax.fori_loop(0, _NG, p1, negv)
        mx = jnp.max(macc)
        thr = mx - 1.0

        def p2(g, base):
            gm = gmax[pl.ds(g * _L, _L)]
            m = gm > thr
            mi = jnp.where(m, 1, 0)
            pos = plsc.cumsum(mi) - mi
            plsc.store_scatter(clist, [pos + base], g * _L + lane, mask=m)
            return base + plsc.all_reduce_population_count(m)

        base = lax.fori_loop(0, _NG, p2, jnp.zeros((_L,), jnp.int32))
        nact = jnp.max(base)
        plsc.store_scatter(clist, [nact + lane], jnp.full((_L,), _NV, jnp.int32))
        ngr = lax.shift_right_logical(nact + (_L - 1), 4)

        def p3(q, c, xv=xv):
            cl = clist[pl.ds(q * _L, _L)]
            bv = lax.shift_right_logical(cl, 4) * 256 + jnp.bitwise_and(cl, 15)
            for j in range(_L):
                av[pl.ds(q * 256 + j * _L, _L)] = plsc.load_gather(
                    xv, [bv + j * _L]
                )
            return c

        lax.fori_loop(0, ngr, p3, 0)

        lo0 = thr + zero
        hi0 = mx + zero

        def p4(_, carry):
            lo, hi, t = carry

            def ev(j, c):
                s, k = c
                for u in range(8):
                    v = av[pl.ds(j * 128 + u * _L, _L)]
                    m = v > t
                    s = s + jnp.where(m, v, 0.0)
                    k = k + jnp.where(m, 1.0, 0.0)
                return s, k

            s_v, k_v = lax.fori_loop(0, ngr * 2, ev, (zero, zero))
            s = jnp.sum(s_v) + zero
            k = jnp.sum(k_v) + zero
            f = s - k * t
            ge = f >= 1.0
            lo = jnp.where(ge, t, lo)
            hi = jnp.where(ge, hi, t)
            nt = jnp.where(k > 0.5, (s - 1.0) / jnp.maximum(k, 1.0), lo)
            lo = jnp.maximum(lo, nt)
            return lo, hi, 0.5 * (lo + hi)

        tau, _hi, _t = lax.fori_loop(0, _T, p4, (lo0, hi0, lo0))

        for j in range(8):
            taubuf[pl.ds(j * _L, _L)] = tau
        pltpu.sync_copy(taubuf, tau_hbm.at[row0 + r])


def _sc_tau(x):
    mesh = plsc.VectorSubcoreMesh(
        core_axis_name="c", subcore_axis_name="s",
        num_cores=_NC, num_subcores=_NS,
    )
    return pl.kernel(
        _sc_tau_body,
        out_type=jax.ShapeDtypeStruct((_ROWS, 128), jnp.float32),
        mesh=mesh,
        scratch_types=[
            pltpu.VMEM((_N + _PAD,), jnp.float32),
            pltpu.VMEM((_N + _PAD,), jnp.float32),
            pltpu.VMEM((_N + _PAD,), jnp.float32),
            pltpu.VMEM((_NV,), jnp.float32),
            pltpu.VMEM((_NV + _L,), jnp.int32),
            pltpu.VMEM((128,), jnp.float32),
            pltpu.SemaphoreType.DMA,
            pltpu.SemaphoreType.DMA,
        ],
        compiler_params=pltpu.CompilerParams(needs_layout_passes=False),
    )(x)


def _tc_relu_body(x_ref, tau_ref, o_ref):
    o_ref[...] = jnp.maximum(x_ref[...] - tau_ref[...][:, 0:1], 0.0)


def _tc_relu(x, tau):
    rows, n = x.shape
    br = 8
    return pl.pallas_call(
        _tc_relu_body,
        grid=(rows // br,),
        in_specs=[
            pl.BlockSpec((br, n), lambda i: (i, 0)),
            pl.BlockSpec((br, 128), lambda i: (i, 0)),
        ],
        out_specs=pl.BlockSpec((br, n), lambda i: (i, 0)),
        out_shape=jax.ShapeDtypeStruct((rows, n), x.dtype),
    )(x, tau)


@jax.jit
def kernel(x):
    return _tc_relu(x, _sc_tau(x))


# speedup vs baseline: 1.0054x; 1.0054x over previous
import jax
import jax.numpy as jnp
from jax import lax
from jax.experimental import pallas as pl
from jax.experimental.pallas import tpu as pltpu
from jax.experimental.pallas import tpu_sc as plsc

_L = 16
_NC = 2
_NS = 16
_NW = _NC * _NS
_ROWS = 128
_N = 32768
_NV = _N // _L
_NG = _NV // _L
_RPW = _ROWS // _NW
_T = 12
_PAD = 256
_NEG = -3.0e38


def _tree_max(vs):
    while len(vs) > 1:
        vs = [jnp.maximum(vs[i], vs[i + 1]) for i in range(0, len(vs) - 1, 2)] + (
            [vs[-1]] if len(vs) % 2 else []
        )
    return vs[0]


def _sc_tau_body(x_hbm, tau_hbm, xva, xvb, av, gmax, clist, taubuf, sia, sib):
    cid = lax.axis_index("c")
    sid = lax.axis_index("s")
    wid = sid * _NC + cid
    row0 = wid * _RPW
    lane = lax.iota(jnp.int32, _L)
    zero = jnp.zeros((_L,), jnp.float32)
    negv = jnp.full((_L,), _NEG, jnp.float32)

    in_descs = [None] * _RPW
    in_descs[0] = pltpu.async_copy(x_hbm.at[row0], xva.at[pl.ds(0, _N)], sia)
    for j in range(_PAD // _L):
        xva[pl.ds(_N + j * _L, _L)] = negv
        xvb[pl.ds(_N + j * _L, _L)] = negv

    for r in range(_RPW):
        xv = xva if r % 2 == 0 else xvb
        in_descs[r].wait()
        if r + 1 < _RPW:
            nxv = xvb if r % 2 == 0 else xva
            nsi = sib if r % 2 == 0 else sia
            in_descs[r + 1] = pltpu.async_copy(
                x_hbm.at[row0 + r + 1], nxv.at[pl.ds(0, _N)], nsi
            )

        def p1(g, macc, xv=xv):
            vs = [xv[pl.ds(g * 256 + j * _L, _L)] for j in range(_L)]
            gm = _tree_max(vs)
            gmax[pl.ds(g * _L, _L)] = gm
            return jnp.maximum(macc, gm)

        macc = lax.fori_loop(0, _NG, p1, negv)
        mx = jnp.max(macc)
        thr = mx - 1.0

        def p2(g, base):
            gm = gmax[pl.ds(g * _L, _L)]
            m = gm > thr
            mi = jnp.where(m, 1, 0)
            pos = plsc.cumsum(mi) - mi
            plsc.store_scatter(clist, [pos + base], g * _L + lane, mask=m)
            return base + plsc.all_reduce_population_count(m)

        base = lax.fori_loop(0, _NG, p2, jnp.zeros((_L,), jnp.int32))
        nact = jnp.max(base)
        plsc.store_scatter(clist, [nact + lane], jnp.full((_L,), _NV, jnp.int32))
        ngr = lax.shift_right_logical(nact + (_L - 1), 4)

        def p3(q, c, xv=xv):
            cl = clist[pl.ds(q * _L, _L)]
            bv = lax.shift_right_logical(cl, 4) * 256 + jnp.bitwise_and(cl, 15)
            for j in range(_L):
                av[pl.ds(q * 256 + j * _L, _L)] = plsc.load_gather(
                    xv, [bv + j * _L]
                )
            return c

        lax.fori_loop(0, ngr, p3, 0)

        lo0 = thr + zero
        hi0 = mx + zero

        def p4(_, carry):
            lo, hi, t = carry

            def ev(j, c):
                s, k = c
                for u in range(8):
                    v = av[pl.ds(j * 128 + u * _L, _L)]
                    m = v > t
                    s = s + jnp.where(m, v, 0.0)
                    k = k + jnp.where(m, 1.0, 0.0)
                return s, k

            s_v, k_v = lax.fori_loop(0, ngr * 2, ev, (zero, zero))
            s = jnp.sum(s_v) + zero
            k = jnp.sum(k_v) + zero
            f = s - k * t
            ge = f >= 1.0
            lo = jnp.where(ge, t, lo)
            hi = jnp.where(ge, hi, t)
            nt = jnp.where(k > 0.5, (s - 1.0) / jnp.maximum(k, 1.0), lo)
            lo = jnp.maximum(lo, nt)
            return lo, hi, 0.5 * (lo + hi)

        tau, _hi, _t = lax.fori_loop(0, _T, p4, (lo0, hi0, lo0))

        for j in range(8):
            taubuf[pl.ds(r * 128 + j * _L, _L)] = tau
    pltpu.sync_copy(taubuf, tau_hbm.at[pl.ds(row0 * 128, _RPW * 128)])


def _sc_tau(x):
    mesh = plsc.VectorSubcoreMesh(
        core_axis_name="c", subcore_axis_name="s",
        num_cores=_NC, num_subcores=_NS,
    )
    return pl.kernel(
        _sc_tau_body,
        out_type=jax.ShapeDtypeStruct((_ROWS * 128,), jnp.float32),
        mesh=mesh,
        scratch_types=[
            pltpu.VMEM((_N + _PAD,), jnp.float32),
            pltpu.VMEM((_N + _PAD,), jnp.float32),
            pltpu.VMEM((_N + _PAD,), jnp.float32),
            pltpu.VMEM((_NV,), jnp.float32),
            pltpu.VMEM((_NV + _L,), jnp.int32),
            pltpu.VMEM((_RPW * 128,), jnp.float32),
            pltpu.SemaphoreType.DMA,
            pltpu.SemaphoreType.DMA,
        ],
        compiler_params=pltpu.CompilerParams(needs_layout_passes=False),
    )(x)


def _tc_relu_body(x_ref, tau_ref, o_ref):
    o_ref[...] = jnp.maximum(x_ref[...] - tau_ref[...][:, 0:1], 0.0)


def _tc_relu(x, tau):
    rows, n = x.shape
    br = 8
    return pl.pallas_call(
        _tc_relu_body,
        grid=(rows // br,),
        in_specs=[
            pl.BlockSpec((br, n), lambda i: (i, 0)),
            pl.BlockSpec((br, 128), lambda i: (i, 0)),
        ],
        out_specs=pl.BlockSpec((br, n), lambda i: (i, 0)),
        out_shape=jax.ShapeDtypeStruct((rows, n), x.dtype),
    )(x, tau)


@jax.jit
def kernel(x):
    return _tc_relu(x, _sc_tau(x).reshape(_ROWS, 128))


# speedup vs baseline: 1.2096x; 1.2032x over previous
import jax
import jax.numpy as jnp
from jax import lax
from jax.experimental import pallas as pl
from jax.experimental.pallas import tpu as pltpu
from jax.experimental.pallas import tpu_sc as plsc

_L = 16
_NC = 2
_NS = 16
_NW = _NC * _NS
_ROWS = 128
_N = 32768
_NH = _N // 2
_NV = _N // _L
_NG = _NV // _L
_RPW = _ROWS // _NW
_T = 10
_PAD = 256
_NEG = -3.0e38


def _tree_max(vs):
    while len(vs) > 1:
        vs = [jnp.maximum(vs[i], vs[i + 1]) for i in range(0, len(vs) - 1, 2)] + (
            [vs[-1]] if len(vs) % 2 else []
        )
    return vs[0]


def _sc_body(x_hbm, out_hbm, xva, xvb, avov, gmax, clist, clist2,
             sia0, sia1, sib0, sib1, so0, so1):
    cid = lax.axis_index("c")
    sid = lax.axis_index("s")
    wid = sid * _NC + cid
    row0 = wid * _RPW
    lane = lax.iota(jnp.int32, _L)
    zero = jnp.zeros((_L,), jnp.float32)
    negv = jnp.full((_L,), _NEG, jnp.float32)

    def start_in(row, buf, s0, s1):
        return (
            pltpu.async_copy(x_hbm.at[row, pl.ds(0, _NH)],
                             buf.at[pl.ds(0, _NH)], s0),
            pltpu.async_copy(x_hbm.at[row, pl.ds(_NH, _NH)],
                             buf.at[pl.ds(_NH, _NH)], s1),
        )

    in_descs = [None] * _RPW
    in_descs[0] = start_in(row0, xva, sia0, sia1)
    for j in range(_PAD // _L):
        xva[pl.ds(_N + j * _L, _L)] = negv
        xvb[pl.ds(_N + j * _L, _L)] = negv

    out_descs = None
    for r in range(_RPW):
        xv = xva if r % 2 == 0 else xvb
        for d in in_descs[r]:
            d.wait()
        if r + 1 < _RPW:
            nxv = xvb if r % 2 == 0 else xva
            s0, s1 = (sib0, sib1) if r % 2 == 0 else (sia0, sia1)
            in_descs[r + 1] = start_in(row0 + r + 1, nxv, s0, s1)

        def p1(g, carry, xv=xv):
            macc, base = carry
            vs = [xv[pl.ds(g * 256 + j * _L, _L)] for j in range(_L)]
            gm = _tree_max(vs)
            gmax[pl.ds(g * _L, _L)] = gm
            macc = jnp.maximum(macc, gm)
            m = gm > macc - 1.0
            mi = jnp.where(m, 1, 0)
            pos = plsc.cumsum(mi) - mi
            plsc.store_scatter(clist, [pos + base], g * _L + lane, mask=m)
            return macc, base + plsc.all_reduce_population_count(m)

        macc, base = lax.fori_loop(
            0, _NG, p1, (negv, jnp.zeros((_L,), jnp.int32))
        )
        mx = jnp.max(macc)
        thr = mx - 1.0
        nraw = jnp.max(base)
        plsc.store_scatter(clist, [nraw + lane], jnp.full((_L,), _NV, jnp.int32))
        gmax[pl.ds(_NV, _L)] = negv
        nrg = lax.shift_right_logical(nraw + (_L - 1), 4)

        def p2(q, base2):
            cl = clist[pl.ds(q * _L, _L)]
            gv = plsc.load_gather(gmax, [cl])
            m = gv > thr
            mi = jnp.where(m, 1, 0)
            pos = plsc.cumsum(mi) - mi
            plsc.store_scatter(clist2, [pos + base2], cl, mask=m)
            return base2 + plsc.all_reduce_population_count(m)

        base2 = lax.fori_loop(0, nrg, p2, jnp.zeros((_L,), jnp.int32))
        nact = jnp.max(base2)
        plsc.store_scatter(clist2, [nact + lane], jnp.full((_L,), _NV, jnp.int32))
        ngr = lax.shift_right_logical(nact + (_L - 1), 4)

        if out_descs is not None:
            for d in out_descs:
                d.wait()

        def p3(q, c, xv=xv):
            cl = clist2[pl.ds(q * _L, _L)]
            bv = lax.shift_right_logical(cl, 4) * 256 + jnp.bitwise_and(cl, 15)
            for j in range(_L):
                avov[pl.ds(q * 256 + j * _L, _L)] = plsc.load_gather(
                    xv, [bv + j * _L]
                )
            return c

        lax.fori_loop(0, ngr, p3, 0)

        lo0 = thr + zero
        hi0 = mx + zero

        def p4(_, carry):
            lo, hi, t = carry

            def ev(j, c):
                s, k = c
                for u in range(8):
                    v = avov[pl.ds(j * 128 + u * _L, _L)]
                    m = v > t
                    s = s + jnp.where(m, v, 0.0)
                    k = k + jnp.where(m, 1.0, 0.0)
                return s, k

            s_v, k_v = lax.fori_loop(0, ngr * 2, ev, (zero, zero))
            s = jnp.sum(s_v) + zero
            k = jnp.sum(k_v) + zero
            f = s - k * t
            ge = f >= 1.0
            lo = jnp.where(ge, t, lo)
            hi = jnp.where(ge, hi, t)
            nt = jnp.where(k > 0.5, (s - 1.0) / jnp.maximum(k, 1.0), lo)
            lo = jnp.maximum(lo, nt)
            return lo, hi, 0.5 * (lo + hi)

        tau, _hi, _t = lax.fori_loop(0, _T, p4, (lo0, hi0, lo0))

        def p5(i, c, xv=xv):
            for u in range(8):
                sl = pl.ds(i * 128 + u * _L, _L)
                avov[sl] = jnp.maximum(xv[sl] - tau, 0.0)
            return c

        lax.fori_loop(0, _NV // 8, p5, 0)
        out_descs = (
            pltpu.async_copy(avov.at[pl.ds(0, _NH)],
                             out_hbm.at[row0 + r, pl.ds(0, _NH)], so0),
            pltpu.async_copy(avov.at[pl.ds(_NH, _NH)],
                             out_hbm.at[row0 + r, pl.ds(_NH, _NH)], so1),
        )
    for d in out_descs:
        d.wait()


@jax.jit
def kernel(x):
    mesh = plsc.VectorSubcoreMesh(
        core_axis_name="c", subcore_axis_name="s",
        num_cores=_NC, num_subcores=_NS,
    )
    return pl.kernel(
        _sc_body,
        out_type=jax.ShapeDtypeStruct((_ROWS, _N), jnp.float32),
        mesh=mesh,
        scratch_types=[
            pltpu.VMEM((_N + _PAD,), jnp.float32),
            pltpu.VMEM((_N + _PAD,), jnp.float32),
            pltpu.VMEM((_N + _PAD,), jnp.float32),
            pltpu.VMEM((_NV + _L,), jnp.float32),
            pltpu.VMEM((_NV + _L,), jnp.int32),
            pltpu.VMEM((_NV + _L,), jnp.int32),
            pltpu.SemaphoreType.DMA,
            pltpu.SemaphoreType.DMA,
            pltpu.SemaphoreType.DMA,
            pltpu.SemaphoreType.DMA,
            pltpu.SemaphoreType.DMA,
            pltpu.SemaphoreType.DMA,
        ],
        compiler_params=pltpu.CompilerParams(needs_layout_passes=False),
    )(x)


# speedup vs baseline: 1.2425x; 1.0272x over previous
import jax
import jax.numpy as jnp
from jax import lax
from jax.experimental import pallas as pl
from jax.experimental.pallas import tpu as pltpu
from jax.experimental.pallas import tpu_sc as plsc

_L = 16
_NC = 2
_NS = 16
_NW = _NC * _NS
_ROWS = 128
_N = 32768
_NH = _N // 2
_NV = _N // _L
_NG = _NV // _L
_RPW = _ROWS // _NW
_T = 12
_PAD = 256
_NEG = -3.0e38


def _tree_max(vs):
    while len(vs) > 1:
        vs = [jnp.maximum(vs[i], vs[i + 1]) for i in range(0, len(vs) - 1, 2)] + (
            [vs[-1]] if len(vs) % 2 else []
        )
    return vs[0]


def _sc_body(x_hbm, out_hbm, xva, xvb, avov, gmax, clist,
             sia0, sia1, sib0, sib1, so0, so1):
    cid = lax.axis_index("c")
    sid = lax.axis_index("s")
    wid = sid * _NC + cid
    row0 = wid * _RPW
    lane = lax.iota(jnp.int32, _L)
    zero = jnp.zeros((_L,), jnp.float32)
    negv = jnp.full((_L,), _NEG, jnp.float32)

    def start_in(row, buf, s0, s1):
        return (
            pltpu.async_copy(x_hbm.at[row, pl.ds(0, _NH)],
                             buf.at[pl.ds(0, _NH)], s0),
            pltpu.async_copy(x_hbm.at[row, pl.ds(_NH, _NH)],
                             buf.at[pl.ds(_NH, _NH)], s1),
        )

    in_descs = [None] * _RPW
    in_descs[0] = start_in(row0, xva, sia0, sia1)
    for j in range(_PAD // _L):
        xva[pl.ds(_N + j * _L, _L)] = negv
        xvb[pl.ds(_N + j * _L, _L)] = negv

    out_descs = None
    for r in range(_RPW):
        xv = xva if r % 2 == 0 else xvb
        for d in in_descs[r]:
            d.wait()
        if r + 1 < _RPW:
            nxv = xvb if r % 2 == 0 else xva
            s0, s1 = (sib0, sib1) if r % 2 == 0 else (sia0, sia1)
            in_descs[r + 1] = start_in(row0 + r + 1, nxv, s0, s1)

        def p1(g, macc, xv=xv):
            vs = [xv[pl.ds(g * 256 + j * _L, _L)] for j in range(_L)]
            gm = _tree_max(vs)
            gmax[pl.ds(g * _L, _L)] = gm
            return jnp.maximum(macc, gm)

        macc = lax.fori_loop(0, _NG, p1, negv)
        mx = jnp.max(macc)
        thr = mx - 1.0

        def p2(g, base):
            gm = gmax[pl.ds(g * _L, _L)]
            m = gm > thr
            mi = jnp.where(m, 1, 0)
            pos = plsc.cumsum(mi) - mi
            plsc.store_scatter(clist, [pos + base], g * _L + lane, mask=m)
            return base + plsc.all_reduce_population_count(m)

        base = lax.fori_loop(0, _NG, p2, jnp.zeros((_L,), jnp.int32))
        nact = jnp.max(base)
        plsc.store_scatter(clist, [nact + lane], jnp.full((_L,), _NV, jnp.int32))
        ngr = lax.shift_right_logical(nact + (_L - 1), 4)

        if out_descs is not None:
            for d in out_descs:
                d.wait()

        def p3(q, c, xv=xv):
            cl = clist[pl.ds(q * _L, _L)]
            bv = lax.shift_right_logical(cl, 4) * 256 + jnp.bitwise_and(cl, 15)
            for j in range(_L):
                avov[pl.ds(q * 256 + j * _L, _L)] = plsc.load_gather(
                    xv, [bv + j * _L]
                )
            return c

        lax.fori_loop(0, ngr, p3, 0)

        lo0 = thr + zero
        hi0 = mx + zero

        def p4(_, carry):
            lo, hi, t = carry

            def ev(j, c):
                s, k = c
                for u in range(8):
                    v = avov[pl.ds(j * 128 + u * _L, _L)]
                    m = v > t
                    s = s + jnp.where(m, v, 0.0)
                    k = k + jnp.where(m, 1.0, 0.0)
                return s, k

            s_v, k_v = lax.fori_loop(0, ngr * 2, ev, (zero, zero))
            s = jnp.sum(s_v) + zero
            k = jnp.sum(k_v) + zero
            f = s - k * t
            ge = f >= 1.0
            lo = jnp.where(ge, t, lo)
            hi = jnp.where(ge, hi, t)
            nt = jnp.where(k > 0.5, (s - 1.0) / jnp.maximum(k, 1.0), lo)
            lo = jnp.maximum(lo, nt)
            return lo, hi, 0.5 * (lo + hi)

        tau, _hi, _t = lax.fori_loop(0, _T, p4, (lo0, hi0, lo0))

        def p5(i, c, xv=xv):
            for u in range(8):
                sl = pl.ds(i * 128 + u * _L, _L)
                avov[sl] = jnp.maximum(xv[sl] - tau, 0.0)
            return c

        lax.fori_loop(0, _NV // 8, p5, 0)
        out_descs = (
            pltpu.async_copy(avov.at[pl.ds(0, _NH)],
                             out_hbm.at[row0 + r, pl.ds(0, _NH)], so0),
            pltpu.async_copy(avov.at[pl.ds(_NH, _NH)],
                             out_hbm.at[row0 + r, pl.ds(_NH, _NH)], so1),
        )
    for d in out_descs:
        d.wait()


@jax.jit
def kernel(x):
    mesh = plsc.VectorSubcoreMesh(
        core_axis_name="c", subcore_axis_name="s",
        num_cores=_NC, num_subcores=_NS,
    )
    return pl.kernel(
        _sc_body,
        out_type=jax.ShapeDtypeStruct((_ROWS, _N), jnp.float32),
        mesh=mesh,
        scratch_types=[
            pltpu.VMEM((_N + _PAD,), jnp.float32),
            pltpu.VMEM((_N + _PAD,), jnp.float32),
            pltpu.VMEM((_N + _PAD,), jnp.float32),
            pltpu.VMEM((_NV,), jnp.float32),
            pltpu.VMEM((_NV + _L,), jnp.int32),
            pltpu.SemaphoreType.DMA,
            pltpu.SemaphoreType.DMA,
            pltpu.SemaphoreType.DMA,
            pltpu.SemaphoreType.DMA,
            pltpu.SemaphoreType.DMA,
            pltpu.SemaphoreType.DMA,
        ],
        compiler_params=pltpu.CompilerParams(needs_layout_passes=False),
    )(x)
